# baseline (speedup 1.0000x reference)
import functools

import jax
import jax.numpy as jnp
from jax import lax
from jax.experimental import pallas as pl
from jax.experimental.pallas import tpu as pltpu
from jax.experimental.pallas import tpu_sc as plsc

_TEMP = 0.1
_EPS = 1e-8
_A = 512
_S = 2048
_N = 50000
_D = 256
_MASK_OFS = 4.0
_MASK_THR = 2.0
_NCLS = 16

_NC = 2
_NS = 16
_NW = _NC * _NS
_PERW = _A // _NW
_LANES = 16

_BN = 2048
_NBLK = (_N + _BN - 1) // _BN


def _gather_anchor_rows(x, y, anchors):
    mesh = plsc.VectorSubcoreMesh(core_axis_name="c", subcore_axis_name="s")

    @functools.partial(
        pl.kernel,
        mesh=mesh,
        out_type=[
            jax.ShapeDtypeStruct((_A, _D), jnp.float32),
            jax.ShapeDtypeStruct((_A,), jnp.int32),
        ],
        scratch_types=[
            pltpu.VMEM((_PERW,), jnp.int32),
            pltpu.VMEM((_PERW, _D), jnp.float32),
            pltpu.VMEM((_PERW,), jnp.int32),
            pltpu.SemaphoreType.DMA,
            pltpu.SemaphoreType.DMA,
        ],
    )
    def k(x_hbm, y_hbm, anc_hbm, xa_out, ya_out, idx_v, rows_v, yv, sem1, sem2):
        wid = lax.axis_index("s") * _NC + lax.axis_index("c")
        base = wid * _PERW
        pltpu.sync_copy(anc_hbm.at[pl.ds(base, _PERW)], idx_v)
        cp1 = pltpu.async_copy(x_hbm.at[idx_v], rows_v, sem1)
        cp2 = pltpu.async_copy(y_hbm.at[idx_v], yv, sem2)
        cp1.wait()
        cp2.wait()
        pltpu.sync_copy(rows_v, xa_out.at[pl.ds(base, _PERW)])
        pltpu.sync_copy(yv, ya_out.at[pl.ds(base, _PERW)])

    return k(x, y, anchors)


def _make_sim_body(ah):
    def _sim_body(xa_ref, ya_ref, x_ref, y_ref, s_ref):
        xa = xa_ref[...]
        na = jnp.sqrt(jnp.sum(xa * xa, axis=1, keepdims=True))
        xan = xa / jnp.maximum(na, _EPS)
        xb = x_ref[...]
        nb = jnp.sqrt(jnp.sum(xb * xb, axis=1, keepdims=True))
        xbn = xb / jnp.maximum(nb, _EPS)
        sim = lax.dot_general(
            xan, xbn, (((1,), (1,)), ((), ())),
            preferred_element_type=jnp.float32)
        m = y_ref[...][None, :] == ya_ref[...]
        s_ref[...] = (sim + jnp.where(m, _MASK_OFS, 0.0)).reshape(ah * _BN)
    return _sim_body


def _build_sim(x, y, xa, ya2, ah):
    return pl.pallas_call(
        _make_sim_body(ah),
        grid=(_NBLK,),
        in_specs=[
            pl.BlockSpec((ah, _D), lambda j: (0, 0)),
            pl.BlockSpec((ah, 1), lambda j: (0, 0)),
            pl.BlockSpec((_BN, _D), lambda j: (j, 0)),
            pl.BlockSpec((_BN,), lambda j: (j,)),
        ],
        out_specs=pl.BlockSpec((ah * _BN,), lambda j: (j,)),
        out_shape=jax.ShapeDtypeStruct((_NBLK * ah * _BN,), jnp.float32),
        compiler_params=pltpu.CompilerParams(
            dimension_semantics=("arbitrary",)),
    )(xa, ya2, x, y)


_CH = 128
_NCH = _S // _CH


def _sample_reduce(s_flat, samples, ah):
    mesh = plsc.VectorSubcoreMesh(core_axis_name="c", subcore_axis_name="s")
    _NBUF = 3
    perw = ah // _NW

    @functools.partial(
        pl.kernel,
        mesh=mesh,
        out_type=[
            jax.ShapeDtypeStruct((ah, _LANES), jnp.float32),
            jax.ShapeDtypeStruct((ah, _LANES), jnp.float32),
            jax.ShapeDtypeStruct((ah, _LANES), jnp.float32),
        ],
        scratch_types=[
            pltpu.VMEM((perw, _S), jnp.int32),
            pltpu.VMEM((_NBUF * _NCH, _CH), jnp.int32),
            pltpu.VMEM((_NBUF * _NCH, _CH), jnp.float32),
            pltpu.VMEM((perw, _LANES), jnp.float32),
            pltpu.VMEM((perw, _LANES), jnp.float32),
            pltpu.VMEM((perw, _LANES), jnp.float32),
            pltpu.SemaphoreType.DMA,
            pltpu.SemaphoreType.DMA,
            pltpu.SemaphoreType.DMA,
        ],
    )
    def k(s_hbm, samp_hbm, num_out, den_out, cnt_out,
          samp_v, gix_v, vals_v, num_v, den_v, cnt_v, sem0, sem1, sem2):
        wid = lax.axis_index("s") * _NC + lax.axis_index("c")
        base = wid * perw
        pltpu.sync_copy(samp_hbm.at[pl.ds(base, perw)], samp_v)
        zero16 = jnp.zeros((_LANES,), jnp.float32)
        per_chunk = _CH // _LANES
        sems = (sem0, sem1, sem2)

        def build(la, buf):
            abase = (base + la) * _BN

            def b(i, _):
                c = i // per_chunk
                o = (i % per_chunk) * _LANES
                s16 = samp_v[la, pl.ds(i * _LANES, _LANES)]
                jb = lax.shift_right_logical(s16, 11)
                jo = jnp.bitwise_and(s16, _BN - 1)
                gix_v[buf * _NCH + c, pl.ds(o, _LANES)] = (
                    jb * (ah * _BN) + jo + abase)
                return 0

            lax.fori_loop(0, _S // _LANES, b, 0)

        def fire(buf):
            return [
                pltpu.async_copy(s_hbm.at[gix_v.at[buf * _NCH + c]],
                                 vals_v.at[buf * _NCH + c], sems[buf])
                for c in range(_NCH)
            ]

        def compute(la, buf):
            def inner(i, carry):
                num, den, cnt = carry
                c = i // per_chunk
                o = (i % per_chunk) * _LANES
                v = vals_v[buf * _NCH + c, pl.ds(o, _LANES)]
                m = v >= _MASK_THR
                e = jnp.exp((v - jnp.where(m, _MASK_OFS, 0.0)) * (1.0 / _TEMP))
                return (num + jnp.where(m, e, 0.0),
                        den + e,
                        cnt + jnp.where(m, 1.0, 0.0))

            num, den, cnt = lax.fori_loop(
                0, _S // _LANES, inner, (zero16, zero16, zero16))
            num_v[la, :] = num
            den_v[la, :] = den
            cnt_v[la, :] = cnt

        inflight = {}
        for la in range(min(_NBUF - 1, perw)):
            build(la, la % _NBUF)
            inflight[la] = fire(la % _NBUF)
        for la in range(perw):
            nf = la + _NBUF - 1
            if nf < perw:
                build(nf, nf % _NBUF)
                inflight[nf] = fire(nf % _NBUF)
            for cp in inflight.pop(la):
                cp.wait()
            compute(la, la % _NBUF)
        pltpu.sync_copy(num_v, num_out.at[pl.ds(base, perw)])
        pltpu.sync_copy(den_v, den_out.at[pl.ds(base, perw)])
        pltpu.sync_copy(cnt_v, cnt_out.at[pl.ds(base, perw)])

    return k(s_flat, samples)


def _final_body(num_ref, den_ref, cnt_ref, out_ref):
    num = jnp.sum(num_ref[...], axis=1)
    den = jnp.sum(den_ref[...], axis=1)
    cnt = jnp.sum(cnt_ref[...], axis=1)
    per = (-1.0 / jnp.maximum(cnt, 1.0)) * jnp.log(num / den)
    out_ref[...] = jnp.sum(per).reshape(1, 1)


def _final_loss(num, den, cnt):
    out = pl.pallas_call(
        _final_body,
        out_shape=jax.ShapeDtypeStruct((1, 1), jnp.float32),
    )(num, den, cnt)
    return out[0, 0]


_NHALF = 2


def kernel(x, y, anchors, samples):
    y = y.astype(jnp.int32)
    anchors = anchors.astype(jnp.int32)
    samples = samples.astype(jnp.int32)
    xa, ya = _gather_anchor_rows(x, y, anchors)
    ah = _A // _NHALF
    parts = []
    for h in range(_NHALF):
        sl = slice(h * ah, (h + 1) * ah)
        s_mat = _build_sim(x, y, xa[sl], ya[sl].reshape(ah, 1), ah)
        parts.append(_sample_reduce(s_mat, samples[sl], ah))
    num = jnp.concatenate([p[0] for p in parts], axis=0)
    den = jnp.concatenate([p[1] for p in parts], axis=0)
    cnt = jnp.concatenate([p[2] for p in parts], axis=0)
    return _final_loss(num, den, cnt)


# speedup vs baseline: 1.2224x; 1.2224x over previous
import functools

import jax
import jax.numpy as jnp
from jax import lax
from jax.experimental import pallas as pl
from jax.experimental.pallas import tpu as pltpu
from jax.experimental.pallas import tpu_sc as plsc

_TEMP = 0.1
_EPS = 1e-8
_A = 512
_S = 2048
_N = 50000
_D = 256
_MASK_OFS = 4.0
_MASK_THR = 2.0
_NCLS = 16

_NC = 2
_NS = 16
_NW = _NC * _NS
_PERW = _A // _NW
_LANES = 16

_BN = 4096
_BN_BITS = 11 + (_BN == 4096)
_NBLK = (_N + _BN - 1) // _BN


def _gather_anchor_rows(x, y, anchors):
    mesh = plsc.VectorSubcoreMesh(core_axis_name="c", subcore_axis_name="s")

    @functools.partial(
        pl.kernel,
        mesh=mesh,
        out_type=[
            jax.ShapeDtypeStruct((_A, _D), jnp.float32),
            jax.ShapeDtypeStruct((_A,), jnp.int32),
        ],
        scratch_types=[
            pltpu.VMEM((_PERW,), jnp.int32),
            pltpu.VMEM((_PERW, _D), jnp.float32),
            pltpu.VMEM((_PERW,), jnp.int32),
            pltpu.SemaphoreType.DMA,
            pltpu.SemaphoreType.DMA,
        ],
    )
    def k(x_hbm, y_hbm, anc_hbm, xa_out, ya_out, idx_v, rows_v, yv, sem1, sem2):
        wid = lax.axis_index("s") * _NC + lax.axis_index("c")
        base = wid * _PERW
        pltpu.sync_copy(anc_hbm.at[pl.ds(base, _PERW)], idx_v)
        cp1 = pltpu.async_copy(x_hbm.at[idx_v], rows_v, sem1)
        cp2 = pltpu.async_copy(y_hbm.at[idx_v], yv, sem2)
        cp1.wait()
        cp2.wait()
        pltpu.sync_copy(rows_v, xa_out.at[pl.ds(base, _PERW)])
        pltpu.sync_copy(yv, ya_out.at[pl.ds(base, _PERW)])

    return k(x, y, anchors)


def _make_sim_body(ah):
    def _sim_body(xa_ref, ya_ref, x_ref, y_ref, s_ref):
        xa = xa_ref[...]
        na = jnp.sqrt(jnp.sum(xa * xa, axis=1, keepdims=True))
        xan = xa / jnp.maximum(na, _EPS)
        xb = x_ref[...]
        nb = jnp.sqrt(jnp.sum(xb * xb, axis=1, keepdims=True))
        xbn = xb / jnp.maximum(nb, _EPS)
        sim = lax.dot_general(
            xan, xbn, (((1,), (1,)), ((), ())),
            preferred_element_type=jnp.float32)
        m = y_ref[...][None, :] == ya_ref[...]
        s_ref[...] = (sim + jnp.where(m, _MASK_OFS, 0.0)).reshape(ah * _BN)
    return _sim_body


def _build_sim(x, y, xa, ya2, ah):
    return pl.pallas_call(
        _make_sim_body(ah),
        grid=(_NBLK,),
        in_specs=[
            pl.BlockSpec((ah, _D), lambda j: (0, 0)),
            pl.BlockSpec((ah, 1), lambda j: (0, 0)),
            pl.BlockSpec((_BN, _D), lambda j: (j, 0)),
            pl.BlockSpec((_BN,), lambda j: (j,)),
        ],
        out_specs=pl.BlockSpec((ah * _BN,), lambda j: (j,)),
        out_shape=jax.ShapeDtypeStruct((_NBLK * ah * _BN,), jnp.float32),
        compiler_params=pltpu.CompilerParams(
            dimension_semantics=("parallel",)),
    )(xa, ya2, x, y)


_CH = 128
_NCH = _S // _CH


def _sample_reduce(s_flat, samples, ah):
    mesh = plsc.VectorSubcoreMesh(core_axis_name="c", subcore_axis_name="s")
    _NBUF = 3
    perw = ah // _NW

    @functools.partial(
        pl.kernel,
        mesh=mesh,
        out_type=[
            jax.ShapeDtypeStruct((ah, _LANES), jnp.float32),
            jax.ShapeDtypeStruct((ah, _LANES), jnp.float32),
            jax.ShapeDtypeStruct((ah, _LANES), jnp.float32),
        ],
        scratch_types=[
            pltpu.VMEM((perw, _S), jnp.int32),
            pltpu.VMEM((_NBUF * _NCH, _CH), jnp.int32),
            pltpu.VMEM((_NBUF * _NCH, _CH), jnp.float32),
            pltpu.VMEM((perw, _LANES), jnp.float32),
            pltpu.VMEM((perw, _LANES), jnp.float32),
            pltpu.VMEM((perw, _LANES), jnp.float32),
            pltpu.SemaphoreType.DMA,
            pltpu.SemaphoreType.DMA,
            pltpu.SemaphoreType.DMA,
        ],
    )
    def k(s_hbm, samp_hbm, num_out, den_out, cnt_out,
          samp_v, gix_v, vals_v, num_v, den_v, cnt_v, sem0, sem1, sem2):
        wid = lax.axis_index("s") * _NC + lax.axis_index("c")
        base = wid * perw
        pltpu.sync_copy(samp_hbm.at[pl.ds(base, perw)], samp_v)
        zero16 = jnp.zeros((_LANES,), jnp.float32)
        per_chunk = _CH // _LANES
        sems = (sem0, sem1, sem2)

        def build(la, buf):
            abase = (base + la) * _BN

            def b(i, _):
                c = i // per_chunk
                o = (i % per_chunk) * _LANES
                s16 = samp_v[la, pl.ds(i * _LANES, _LANES)]
                jb = lax.shift_right_logical(s16, _BN_BITS)
                jo = jnp.bitwise_and(s16, _BN - 1)
                gix_v[buf * _NCH + c, pl.ds(o, _LANES)] = (
                    jb * (ah * _BN) + jo + abase)
                return 0

            lax.fori_loop(0, _S // _LANES, b, 0)

        def fire(buf):
            return [
                pltpu.async_copy(s_hbm.at[gix_v.at[buf * _NCH + c]],
                                 vals_v.at[buf * _NCH + c], sems[buf])
                for c in range(_NCH)
            ]

        def compute(la, buf):
            def inner(i, carry):
                num, den, cnt = carry
                c = i // per_chunk
                o = (i % per_chunk) * _LANES
                v = vals_v[buf * _NCH + c, pl.ds(o, _LANES)]
                m = v >= _MASK_THR
                e = jnp.exp((v - jnp.where(m, _MASK_OFS, 0.0)) * (1.0 / _TEMP))
                return (num + jnp.where(m, e, 0.0),
                        den + e,
                        cnt + jnp.where(m, 1.0, 0.0))

            num, den, cnt = lax.fori_loop(
                0, _S // _LANES, inner, (zero16, zero16, zero16))
            num_v[la, :] = num
            den_v[la, :] = den
            cnt_v[la, :] = cnt

        inflight = {}
        for la in range(min(_NBUF - 1, perw)):
            build(la, la % _NBUF)
            inflight[la] = fire(la % _NBUF)
        for la in range(perw):
            nf = la + _NBUF - 1
            if nf < perw:
                build(nf, nf % _NBUF)
                inflight[nf] = fire(nf % _NBUF)
            for cp in inflight.pop(la):
                cp.wait()
            compute(la, la % _NBUF)
        pltpu.sync_copy(num_v, num_out.at[pl.ds(base, perw)])
        pltpu.sync_copy(den_v, den_out.at[pl.ds(base, perw)])
        pltpu.sync_copy(cnt_v, cnt_out.at[pl.ds(base, perw)])

    return k(s_flat, samples)


def _final_body(num_ref, den_ref, cnt_ref, out_ref):
    num = jnp.sum(num_ref[...], axis=1)
    den = jnp.sum(den_ref[...], axis=1)
    cnt = jnp.sum(cnt_ref[...], axis=1)
    per = (-1.0 / jnp.maximum(cnt, 1.0)) * jnp.log(num / den)
    out_ref[...] = jnp.sum(per).reshape(1, 1)


def _final_loss(num, den, cnt):
    out = pl.pallas_call(
        _final_body,
        out_shape=jax.ShapeDtypeStruct((1, 1), jnp.float32),
    )(num, den, cnt)
    return out[0, 0]


_NHALF = 1


def kernel(x, y, anchors, samples):
    y = y.astype(jnp.int32)
    anchors = anchors.astype(jnp.int32)
    samples = samples.astype(jnp.int32)
    xa, ya = _gather_anchor_rows(x, y, anchors)
    ah = _A // _NHALF
    parts = []
    for h in range(_NHALF):
        sl = slice(h * ah, (h + 1) * ah)
        s_mat = _build_sim(x, y, xa[sl], ya[sl].reshape(ah, 1), ah)
        parts.append(_sample_reduce(s_mat, samples[sl], ah))
    num = jnp.concatenate([p[0] for p in parts], axis=0)
    den = jnp.concatenate([p[1] for p in parts], axis=0)
    cnt = jnp.concatenate([p[2] for p in parts], axis=0)
    return _final_loss(num, den, cnt)


# speedup vs baseline: 1.2322x; 1.0080x over previous
import functools

import jax
import jax.numpy as jnp
from jax import lax
from jax.experimental import pallas as pl
from jax.experimental.pallas import tpu as pltpu
from jax.experimental.pallas import tpu_sc as plsc

_TEMP = 0.1
_EPS = 1e-8
_A = 512
_S = 2048
_N = 50000
_D = 256
_MASK_OFS = 4.0
_MASK_THR = 2.0

_NC = 2
_NS = 16
_NW = _NC * _NS
_PERW = _A // _NW
_LANES = 16

_BN = 4096
_BN_BITS = _BN.bit_length() - 1
_NBLK = (_N + _BN - 1) // _BN


def _gather_anchor_rows(x, y, anchors):
    mesh = plsc.VectorSubcoreMesh(core_axis_name="c", subcore_axis_name="s")

    @functools.partial(
        pl.kernel,
        mesh=mesh,
        out_type=[
            jax.ShapeDtypeStruct((_A, _D), jnp.float32),
            jax.ShapeDtypeStruct((_A,), jnp.int32),
        ],
        scratch_types=[
            pltpu.VMEM((_PERW,), jnp.int32),
            pltpu.VMEM((_PERW, _D), jnp.float32),
            pltpu.VMEM((_PERW,), jnp.int32),
            pltpu.SemaphoreType.DMA,
            pltpu.SemaphoreType.DMA,
        ],
    )
    def k(x_hbm, y_hbm, anc_hbm, xa_out, ya_out, idx_v, rows_v, yv, sem1, sem2):
        wid = lax.axis_index("s") * _NC + lax.axis_index("c")
        base = wid * _PERW
        pltpu.sync_copy(anc_hbm.at[pl.ds(base, _PERW)], idx_v)
        cp1 = pltpu.async_copy(x_hbm.at[idx_v], rows_v, sem1)
        cp2 = pltpu.async_copy(y_hbm.at[idx_v], yv, sem2)
        cp1.wait()
        cp2.wait()
        pltpu.sync_copy(rows_v, xa_out.at[pl.ds(base, _PERW)])
        pltpu.sync_copy(yv, ya_out.at[pl.ds(base, _PERW)])

    return k(x, y, anchors)


def _make_sim_body(ah):
    def _sim_body(xa_ref, ya_ref, x_ref, y_ref, s_ref):
        xa = xa_ref[...]
        na = jnp.sqrt(jnp.sum(xa * xa, axis=1, keepdims=True))
        xan = xa / jnp.maximum(na, _EPS)
        xb = x_ref[...]
        nb2 = lax.dot_general(
            jnp.ones((8, _D), jnp.float32), xb * xb,
            (((1,), (1,)), ((), ())),
            preferred_element_type=jnp.float32)[0:1, :]
        invb = 1.0 / jnp.maximum(jnp.sqrt(nb2), _EPS)
        sim = lax.dot_general(
            xan, xb, (((1,), (1,)), ((), ())),
            preferred_element_type=jnp.float32)
        sim = sim * invb
        m = y_ref[...][None, :] == ya_ref[...]
        s_ref[...] = (sim + jnp.where(m, _MASK_OFS, 0.0)).reshape(ah * _BN)
    return _sim_body


def _build_sim(x, y, xa, ya2, ah):
    return pl.pallas_call(
        _make_sim_body(ah),
        grid=(_NBLK,),
        in_specs=[
            pl.BlockSpec((ah, _D), lambda j: (0, 0)),
            pl.BlockSpec((ah, 1), lambda j: (0, 0)),
            pl.BlockSpec((_BN, _D), lambda j: (j, 0)),
            pl.BlockSpec((_BN,), lambda j: (j,)),
        ],
        out_specs=pl.BlockSpec((ah * _BN,), lambda j: (j,)),
        out_shape=jax.ShapeDtypeStruct((_NBLK * ah * _BN,), jnp.float32),
        compiler_params=pltpu.CompilerParams(
            dimension_semantics=("parallel",)),
    )(xa, ya2, x, y)


_CH = 128
_NCH = _S // _CH


def _sample_reduce(s_flat, samples, ah):
    mesh = plsc.VectorSubcoreMesh(core_axis_name="c", subcore_axis_name="s")
    _NBUF = 3
    perw = ah // _NW

    @functools.partial(
        pl.kernel,
        mesh=mesh,
        out_type=[
            jax.ShapeDtypeStruct((ah, _LANES), jnp.float32),
            jax.ShapeDtypeStruct((ah, _LANES), jnp.float32),
            jax.ShapeDtypeStruct((ah, _LANES), jnp.float32),
        ],
        scratch_types=[
            pltpu.VMEM((perw, _S), jnp.int32),
            pltpu.VMEM((_NBUF * _NCH, _CH), jnp.int32),
            pltpu.VMEM((_NBUF * _NCH, _CH), jnp.float32),
            pltpu.VMEM((perw, _LANES), jnp.float32),
            pltpu.VMEM((perw, _LANES), jnp.float32),
            pltpu.VMEM((perw, _LANES), jnp.float32),
            pltpu.SemaphoreType.DMA,
            pltpu.SemaphoreType.DMA,
            pltpu.SemaphoreType.DMA,
        ],
    )
    def k(s_hbm, samp_hbm, num_out, den_out, cnt_out,
          samp_v, gix_v, vals_v, num_v, den_v, cnt_v, sem0, sem1, sem2):
        wid = lax.axis_index("s") * _NC + lax.axis_index("c")
        base = wid * perw
        pltpu.sync_copy(samp_hbm.at[pl.ds(base, perw)], samp_v)
        zero16 = jnp.zeros((_LANES,), jnp.float32)
        per_chunk = _CH // _LANES
        sems = (sem0, sem1, sem2)

        def build(la, buf):
            abase = (base + la) * _BN

            def b(i, _):
                c = i // per_chunk
                o = (i % per_chunk) * _LANES
                s16 = samp_v[la, pl.ds(i * _LANES, _LANES)]
                jb = lax.shift_right_logical(s16, _BN_BITS)
                jo = jnp.bitwise_and(s16, _BN - 1)
                gix_v[buf * _NCH + c, pl.ds(o, _LANES)] = (
                    jb * (ah * _BN) + jo + abase)
                return 0

            lax.fori_loop(0, _S // _LANES, b, 0)

        def fire(buf):
            return [
                pltpu.async_copy(s_hbm.at[gix_v.at[buf * _NCH + c]],
                                 vals_v.at[buf * _NCH + c], sems[buf])
                for c in range(_NCH)
            ]

        def compute(la, buf):
            def inner(i, carry):
                num, den, cnt = carry
                c = i // per_chunk
                o = (i % per_chunk) * _LANES
                v = vals_v[buf * _NCH + c, pl.ds(o, _LANES)]
                m = v >= _MASK_THR
                e = jnp.exp((v - jnp.where(m, _MASK_OFS, 0.0)) * (1.0 / _TEMP))
                return (num + jnp.where(m, e, 0.0),
                        den + e,
                        cnt + jnp.where(m, 1.0, 0.0))

            num, den, cnt = lax.fori_loop(
                0, _S // _LANES, inner, (zero16, zero16, zero16))
            num_v[la, :] = num
            den_v[la, :] = den
            cnt_v[la, :] = cnt

        inflight = {}
        for la in range(min(_NBUF - 1, perw)):
            build(la, la % _NBUF)
            inflight[la] = fire(la % _NBUF)
        for la in range(perw):
            nf = la + _NBUF - 1
            if nf < perw:
                build(nf, nf % _NBUF)
                inflight[nf] = fire(nf % _NBUF)
            for cp in inflight.pop(la):
                cp.wait()
            compute(la, la % _NBUF)
        pltpu.sync_copy(num_v, num_out.at[pl.ds(base, perw)])
        pltpu.sync_copy(den_v, den_out.at[pl.ds(base, perw)])
        pltpu.sync_copy(cnt_v, cnt_out.at[pl.ds(base, perw)])

    return k(s_flat, samples)


def _final_body(num_ref, den_ref, cnt_ref, out_ref):
    num = jnp.sum(num_ref[...], axis=1)
    den = jnp.sum(den_ref[...], axis=1)
    cnt = jnp.sum(cnt_ref[...], axis=1)
    per = (-1.0 / jnp.maximum(cnt, 1.0)) * jnp.log(num / den)
    out_ref[...] = jnp.sum(per).reshape(1, 1)


def _final_loss(num, den, cnt):
    out = pl.pallas_call(
        _final_body,
        out_shape=jax.ShapeDtypeStruct((1, 1), jnp.float32),
    )(num, den, cnt)
    return out[0, 0]


_NHALF = 1


def kernel(x, y, anchors, samples):
    y = y.astype(jnp.int32)
    anchors = anchors.astype(jnp.int32)
    samples = samples.astype(jnp.int32)
    xa, ya = _gather_anchor_rows(x, y, anchors)
    ah = _A // _NHALF
    parts = []
    for h in range(_NHALF):
        sl = slice(h * ah, (h + 1) * ah)
        s_mat = _build_sim(x, y, xa[sl], ya[sl].reshape(ah, 1), ah)
        parts.append(_sample_reduce(s_mat, samples[sl], ah))
    num = jnp.concatenate([p[0] for p in parts], axis=0)
    den = jnp.concatenate([p[1] for p in parts], axis=0)
    cnt = jnp.concatenate([p[2] for p in parts], axis=0)
    return _final_loss(num, den, cnt)
